# Initial kernel scaffold; baseline (speedup 1.0000x reference)
import jax
import jax.numpy as jnp
from jax.experimental import pallas as pl


def kernel(features, same_index, diff_index, edge_index, same_index2, diff_index2, c1_Wq0, c1_bq0, c1_Wk0, c1_bk0, c1_Wv0, c1_bv0, c1_Ws0, c1_bs0, c1_Wq, c1_bq, c1_Wk, c1_bk, c1_Wv, c1_bv, c1_Ws, c1_bs, c2_Wq0, c2_bq0, c2_Wk0, c2_bk0, c2_Wv0, c2_bv0, c2_Ws0, c2_bs0, c2_Wq, c2_bq, c2_Wk, c2_bk, c2_Wv, c2_bv, c2_Ws, c2_bs, c3_Wq, c3_bq, c3_Wk, c3_bk, c3_Wv, c3_bv, c3_Ws, c3_bs, fc_W, fc_b):
    raise NotImplementedError("write your pallas kernel here")


# speedup vs baseline: 1.0034x; 1.0034x over previous
import jax
import jax.numpy as jnp
import numpy as np
from jax.experimental import pallas as pl


def _tconv(x, ei, Wq, bq, Wk, bk, Wv, bv, Ws, bs):
    n = x.shape[0]
    q = x @ Wq + bq
    k = x @ Wk + bk
    v = x @ Wv + bv
    src = ei[0]
    dst = ei[1]
    logits = jnp.sum(q[dst] * k[src], axis=-1) / jnp.sqrt(jnp.float32(q.shape[1]))
    m = jax.ops.segment_max(logits, dst, num_segments=n)
    m = jnp.where(jnp.isfinite(m), m, 0.0)
    e = jnp.exp(logits - m[dst])
    s = jax.ops.segment_sum(e, dst, num_segments=n)
    alpha = e / (s[dst] + 1e-16)
    agg = jax.ops.segment_sum(alpha[:, None] * v[src], dst, num_segments=n)
    return agg + x @ Ws + bs


def _bn(x):
    mu = jnp.mean(x, axis=0)
    var = jnp.var(x, axis=0)
    return (x - mu) / jnp.sqrt(var + 1e-5)


def _final_fc(fc, W, b):
    def body(fc_ref, w_ref, b_ref, o_ref):
        o_ref[...] = fc_ref[...] @ w_ref[...] + b_ref[...]
    return pl.pallas_call(
        body,
        out_shape=jax.ShapeDtypeStruct((fc.shape[0], W.shape[1]), jnp.float32),
    )(fc, W, b)


def kernel(features, same_index, diff_index, edge_index, same_index2, diff_index2,
           c1_Wq0, c1_bq0, c1_Wk0, c1_bk0, c1_Wv0, c1_bv0, c1_Ws0, c1_bs0,
           c1_Wq, c1_bq, c1_Wk, c1_bk, c1_Wv, c1_bv, c1_Ws, c1_bs,
           c2_Wq0, c2_bq0, c2_Wk0, c2_bk0, c2_Wv0, c2_bv0, c2_Ws0, c2_bs0,
           c2_Wq, c2_bq, c2_Wk, c2_bk, c2_Wv, c2_bv, c2_Ws, c2_bs,
           c3_Wq, c3_bq, c3_Wk, c3_bk, c3_Wv, c3_bv, c3_Ws, c3_bs,
           fc_W, fc_b):
    lr = lambda z: jax.nn.leaky_relu(z, 0.01)
    x1 = lr(_bn(_tconv(features, same_index2, c1_Wq0, c1_bq0, c1_Wk0, c1_bk0, c1_Wv0, c1_bv0, c1_Ws0, c1_bs0)))
    x2 = lr(_bn(_tconv(features, diff_index2, c2_Wq0, c2_bq0, c2_Wk0, c2_bk0, c2_Wv0, c2_bv0, c2_Ws0, c2_bs0)))
    x = jnp.concatenate([x1, x2], axis=-1)
    x = lr(_bn(_tconv(x, edge_index, c3_Wq[0], c3_bq[0], c3_Wk[0], c3_bk[0], c3_Wv[0], c3_bv[0], c3_Ws[0], c3_bs[0])))
    fc = x
    for l in range(1, 5):
        x3 = jnp.concatenate([x1, x2], axis=-1)
        x4 = jnp.concatenate([x2, x1], axis=-1)
        x1 = lr(_bn(_tconv(x3, same_index2, c1_Wq[l - 1], c1_bq[l - 1], c1_Wk[l - 1], c1_bk[l - 1], c1_Wv[l - 1], c1_bv[l - 1], c1_Ws[l - 1], c1_bs[l - 1])))
        x2 = lr(_bn(_tconv(x4, diff_index2, c2_Wq[l - 1], c2_bq[l - 1], c2_Wk[l - 1], c2_bk[l - 1], c2_Wv[l - 1], c2_bv[l - 1], c2_Ws[l - 1], c2_bs[l - 1])))
        x = jnp.concatenate([x1, x2], axis=-1)
        x = lr(_bn(_tconv(x, edge_index, c3_Wq[l], c3_bq[l], c3_Wk[l], c3_bk[l], c3_Wv[l], c3_bv[l], c3_Ws[l], c3_bs[l])))
        fc = jnp.concatenate([fc, x], axis=-1)
    return _final_fc(fc, fc_W, fc_b)


# speedup vs baseline: 22.3711x; 22.2962x over previous
import math

import jax
import jax.numpy as jnp
from jax import lax
from jax.experimental import pallas as pl
from jax.experimental.pallas import tpu as pltpu
from jax.experimental.pallas import tpu_sc as plsc

_N = 10000
_E = 640000
_D = 20
_NP = 10240
_C = 128
_CHUNKS = 157
_EP = 32 * _CHUNKS * _C
_RPT = _NP // 16
_INV = 1.0 / math.sqrt(20.0)
_f32 = jnp.float32


def _sc_attn(qs, kvs, srcs, dsts, zeros_pad):
    nconv = len(qs)
    mesh = plsc.VectorSubcoreMesh(core_axis_name="c", subcore_axis_name="s")
    scratch = [pltpu.VMEM_SHARED((_NP, 24), _f32) for _ in range(nconv)]
    scratch += [
        pltpu.VMEM((_C,), jnp.int32),
        pltpu.VMEM((_C,), jnp.int32),
        pltpu.VMEM((_C, 32), _f32),
        pltpu.VMEM((_C, 48), _f32),
        pltpu.VMEM((_C, 24), _f32),
        pltpu.SemaphoreType.DMA,
    ]

    def body(*refs):
        ins = refs[: 4 * nconv + 1]
        outs = refs[4 * nconv + 1: 5 * nconv + 1]
        scr = refs[5 * nconv + 1:]
        accs = scr[:nconv]
        dsti, srci, qbuf, kvbuf, obuf, sem = scr[nconv:]
        c = lax.axis_index("c")
        s = lax.axis_index("s")
        g = c * 16 + s
        zeros_ref = ins[4 * nconv]
        for v in range(nconv):
            pltpu.sync_copy(zeros_ref.at[pl.ds(s * _RPT, _RPT)],
                            accs[v].at[pl.ds(s * _RPT, _RPT)])
        plsc.subcore_barrier()
        iota16 = lax.iota(jnp.int32, 16)
        for v in range(nconv):
            qh, kvh, srch, dsth = ins[4 * v: 4 * v + 4]
            acc = accs[v]

            def chunk(i, carry, qh=qh, kvh=kvh, srch=srch, dsth=dsth, acc=acc):
                base = (g * _CHUNKS + i) * _C
                pltpu.sync_copy(srch.at[pl.ds(base, _C)], srci)
                pltpu.sync_copy(dsth.at[pl.ds(base, _C)], dsti)
                pltpu.async_copy(qh.at[dsti], qbuf, sem).wait()
                pltpu.async_copy(kvh.at[srci], kvbuf, sem).wait()

                def group(gi, cc):
                    eids = gi * 16 + iota16
                    a = jnp.zeros((16,), _f32)
                    for d in range(_D):
                        dv = jnp.full((16,), d, jnp.int32)
                        a = a + (plsc.load_gather(qbuf, [eids, dv])
                                 * plsc.load_gather(kvbuf, [eids, dv]))
                    e = jnp.exp(a * _INV)
                    for d in range(_D):
                        vd = plsc.load_gather(
                            kvbuf, [eids, jnp.full((16,), 24 + d, jnp.int32)])
                        plsc.store_scatter(
                            obuf, [eids, jnp.full((16,), d, jnp.int32)], vd * e)
                    plsc.store_scatter(
                        obuf, [eids, jnp.full((16,), _D, jnp.int32)], e)
                    return cc

                lax.fori_loop(0, _C // 16, group, 0)
                pltpu.sync_copy(obuf, acc.at[dsti], add=True)
                return carry

            lax.fori_loop(0, _CHUNKS, chunk, 0)
        plsc.subcore_barrier()
        for v in range(nconv):
            pltpu.sync_copy(accs[v].at[pl.ds(s * _RPT, _RPT)],
                            outs[v].at[c, pl.ds(s * _RPT, _RPT)])

    fn = pl.kernel(
        body,
        out_type=[jax.ShapeDtypeStruct((2, _NP, 24), _f32)] * nconv,
        mesh=mesh,
        scratch_types=scratch,
        compiler_params=pltpu.CompilerParams(
            needs_layout_passes=False, use_tc_tiling_on_sc=False),
    )
    args = []
    for i in range(nconv):
        args += [qs[i], kvs[i], srcs[i], dsts[i]]
    args.append(zeros_pad)
    res = fn(*args)
    if not isinstance(res, (list, tuple)):
        res = [res]
    return list(res)


def _lr(z):
    return jnp.where(z >= 0, z, 0.01 * z)


def _pre(a0, a1, xs):
    u = a0[:, 0:20] + a1[:, 0:20]
    ssum = a0[:, 20:21] + a1[:, 20:21]
    return u / (ssum + 1e-16) + xs


def _norm_lr(pre, stats):
    mu = stats[0:1, :] * (1.0 / _N)
    var = stats[1:2, :] * (1.0 / _N) - mu * mu
    return _lr((pre - mu) * lax.rsqrt(var + 1e-5))


def _split_proj(o, base, q_ref, kv_ref, xs_ref):
    n = o.shape[0]
    z12 = jnp.zeros((n, 12), _f32)
    z4 = jnp.zeros((n, 4), _f32)
    q = o[:, base: base + 20]
    k = o[:, base + 20: base + 40]
    v = o[:, base + 40: base + 60]
    s = o[:, base + 60: base + 80]
    q_ref[...] = jnp.concatenate([q, z12], axis=1)
    kv_ref[...] = jnp.concatenate([k, z4, v, z4], axis=1)
    xs_ref[...] = s


def _tc_first(features, wcat, bcat):
    R = 1000

    def body(x_ref, w_ref, b_ref, q1, kv1, s1, q2, kv2, s2):
        o = jnp.dot(x_ref[...], w_ref[...],
                    preferred_element_type=_f32) + b_ref[...]
        _split_proj(o, 0, q1, kv1, s1)
        _split_proj(o, 80, q2, kv2, s2)

    outs = [jax.ShapeDtypeStruct((_N, 32), _f32),
            jax.ShapeDtypeStruct((_N, 48), _f32),
            jax.ShapeDtypeStruct((_N, 20), _f32)] * 2
    ospec = [pl.BlockSpec((R, 32), lambda i: (i, 0)),
             pl.BlockSpec((R, 48), lambda i: (i, 0)),
             pl.BlockSpec((R, 20), lambda i: (i, 0))] * 2
    return pl.pallas_call(
        body,
        grid=(_N // R,),
        in_specs=[pl.BlockSpec((R, 2000), lambda i: (i, 0)),
                  pl.BlockSpec((2000, 160), lambda i: (0, 0)),
                  pl.BlockSpec((1, 160), lambda i: (0, 0))],
        out_specs=ospec,
        out_shape=outs,
    )(features, wcat, bcat)


_NB = 10
_R = _N // _NB


def _tc_level(cC, cA, cB, projs):
    has_c = cC is not None
    nproj = len(projs)
    ncomb = 3 if has_c else 2

    def body(*refs):
        ins = refs[: 3 * ncomb + 2 * nproj]
        orefs = refs[3 * ncomb + 2 * nproj: 3 * ncomb + 2 * nproj
                     + (1 if has_c else 0) + 3 * nproj]
        scr = refs[3 * ncomb + 2 * nproj + (1 if has_c else 0) + 3 * nproj:]
        pres = scr[:ncomb]
        stats = scr[ncomb: 2 * ncomb]
        wrefs = ins[3 * ncomb:]
        phase = pl.program_id(0)
        j = pl.program_id(1)

        @pl.when(phase == 0)
        def _():
            for ci in range(ncomb):
                a0, a1, xs = ins[3 * ci: 3 * ci + 3]
                p = _pre(a0[...], a1[...], xs[...])
                pres[ci][pl.ds(j * _R, _R), :] = p
                st = jnp.concatenate(
                    [jnp.sum(p, axis=0, keepdims=True),
                     jnp.sum(p * p, axis=0, keepdims=True)], axis=0)

                @pl.when(j == 0)
                def _(ci=ci, st=st):
                    stats[ci][...] = st

                @pl.when(j > 0)
                def _(ci=ci, st=st):
                    stats[ci][...] = stats[ci][...] + st

        @pl.when(phase == 1)
        def _():
            xall = [_norm_lr(pres[ci][pl.ds(j * _R, _R), :],
                             stats[ci][...]) for ci in range(ncomb)]
            oi = 0
            if has_c:
                orefs[0][...] = xall[0]
                oi = 1
            x1, x2 = xall[-2], xall[-1]
            x12 = jnp.concatenate([x1, x2], axis=1)
            x21 = jnp.concatenate([x2, x1], axis=1)
            for p in range(nproj):
                xin = x12 if projs[p][2] == 'c12' else x21
                o = jnp.dot(xin, wrefs[2 * p][...],
                            preferred_element_type=_f32) + wrefs[2 * p + 1][...]
                _split_proj(o, 0, orefs[oi], orefs[oi + 1], orefs[oi + 2])
                oi += 3

    args = []
    if has_c:
        args += list(cC)
    args += list(cA) + list(cB)
    for w, b, _k in projs:
        args += [w, b]
    blk = lambda r, c: pl.BlockSpec((r, c), lambda p, j: (j, 0))
    full = lambda r, c: pl.BlockSpec((r, c), lambda p, j: (0, 0))
    in_specs = []
    for _ in range(ncomb):
        in_specs += [blk(_R, 24), blk(_R, 24), blk(_R, 20)]
    for _ in range(nproj):
        in_specs += [full(40, 80), full(1, 80)]
    outs = ([jax.ShapeDtypeStruct((_N, 20), _f32)] if has_c else [])
    out_specs = ([blk(_R, 20)] if has_c else [])
    for _ in range(nproj):
        outs += [jax.ShapeDtypeStruct((_N, 32), _f32),
                 jax.ShapeDtypeStruct((_N, 48), _f32),
                 jax.ShapeDtypeStruct((_N, 20), _f32)]
        out_specs += [blk(_R, 32), blk(_R, 48), blk(_R, 20)]
    scratch = ([pltpu.VMEM((_N, 20), _f32)] * ncomb
               + [pltpu.VMEM((2, 20), _f32)] * ncomb)
    return pl.pallas_call(
        body, grid=(2, _NB), in_specs=in_specs, out_specs=out_specs,
        out_shape=outs, scratch_shapes=scratch,
    )(*args)


def _tc_final(a30, a31, xs3, xls, fc_w, fc_b):
    def body(a0, a1, x3, x0, x1, x2, x3l, wr, br, out, pre_s, stat_s):
        phase = pl.program_id(0)
        j = pl.program_id(1)

        @pl.when(phase == 0)
        def _():
            p = _pre(a0[...], a1[...], x3[...])
            pre_s[pl.ds(j * _R, _R), :] = p
            st = jnp.concatenate(
                [jnp.sum(p, axis=0, keepdims=True),
                 jnp.sum(p * p, axis=0, keepdims=True)], axis=0)

            @pl.when(j == 0)
            def _():
                stat_s[...] = st

            @pl.when(j > 0)
            def _():
                stat_s[...] = stat_s[...] + st

        @pl.when(phase == 1)
        def _():
            xl4 = _norm_lr(pre_s[pl.ds(j * _R, _R), :], stat_s[...])
            fc = jnp.concatenate(
                [x0[...], x1[...], x2[...], x3l[...], xl4], axis=1)
            out[...] = jnp.dot(fc, wr[...],
                               preferred_element_type=_f32) + br[...]

    blk = lambda r, c: pl.BlockSpec((r, c), lambda p, j: (j, 0))
    full = lambda r, c: pl.BlockSpec((r, c), lambda p, j: (0, 0))
    return pl.pallas_call(
        body, grid=(2, _NB),
        in_specs=[blk(_R, 24), blk(_R, 24), blk(_R, 20)]
        + [blk(_R, 20)] * 4 + [full(100, 2), full(1, 2)],
        out_specs=blk(_R, 2),
        out_shape=jax.ShapeDtypeStruct((_N, 2), _f32),
        scratch_shapes=[pltpu.VMEM((_N, 20), _f32),
                        pltpu.VMEM((2, 20), _f32)],
    )(a30, a31, xs3, xls[0], xls[1], xls[2], xls[3], fc_w,
      fc_b[None, :])


def _prep_edges(ei):
    pad = jnp.full((_EP - _E,), _N, jnp.int32)
    src = jnp.concatenate([ei[0].astype(jnp.int32), pad])
    dst = jnp.concatenate([ei[1].astype(jnp.int32), pad])
    return src, dst


def _pad_nodes(t):
    return jnp.concatenate(
        [t, jnp.zeros((_NP - _N, t.shape[1]), _f32)], axis=0)


def _wcat(wq, bq, wk, bk, wv, bv, ws, bs):
    w = jnp.concatenate([wq, wk, wv, ws], axis=1)
    b = jnp.concatenate([bq, bk, bv, bs])[None, :]
    return w, b


def _acc_split(acc):
    return acc[0, :_N, :], acc[1, :_N, :]


def kernel(features, same_index, diff_index, edge_index, same_index2, diff_index2,
           c1_Wq0, c1_bq0, c1_Wk0, c1_bk0, c1_Wv0, c1_bv0, c1_Ws0, c1_bs0,
           c1_Wq, c1_bq, c1_Wk, c1_bk, c1_Wv, c1_bv, c1_Ws, c1_bs,
           c2_Wq0, c2_bq0, c2_Wk0, c2_bk0, c2_Wv0, c2_bv0, c2_Ws0, c2_bs0,
           c2_Wq, c2_bq, c2_Wk, c2_bk, c2_Wv, c2_bv, c2_Ws, c2_bs,
           c3_Wq, c3_bq, c3_Wk, c3_bk, c3_Wv, c3_bv, c3_Ws, c3_bs,
           fc_W, fc_b):
    w1, b1 = _wcat(c1_Wq0, c1_bq0, c1_Wk0, c1_bk0, c1_Wv0, c1_bv0, c1_Ws0, c1_bs0)
    w2, b2 = _wcat(c2_Wq0, c2_bq0, c2_Wk0, c2_bk0, c2_Wv0, c2_bv0, c2_Ws0, c2_bs0)
    wcat = jnp.concatenate([w1, w2], axis=1)
    bcat = jnp.concatenate([b1, b2], axis=1)
    wl1 = [_wcat(c1_Wq[l], c1_bq[l], c1_Wk[l], c1_bk[l],
                 c1_Wv[l], c1_bv[l], c1_Ws[l], c1_bs[l]) for l in range(4)]
    wl2 = [_wcat(c2_Wq[l], c2_bq[l], c2_Wk[l], c2_bk[l],
                 c2_Wv[l], c2_bv[l], c2_Ws[l], c2_bs[l]) for l in range(4)]
    wl3 = [_wcat(c3_Wq[l], c3_bq[l], c3_Wk[l], c3_bk[l],
                 c3_Wv[l], c3_bv[l], c3_Ws[l], c3_bs[l]) for l in range(5)]

    srcS, dstS = _prep_edges(same_index2)
    srcD, dstD = _prep_edges(diff_index2)
    srcE, dstE = _prep_edges(edge_index)
    zeros_pad = jnp.zeros((_NP, 24), _f32)

    q1, kv1, s1, q2, kv2, s2 = _tc_first(features, wcat, bcat)
    accs = _sc_attn([_pad_nodes(q1), _pad_nodes(q2)],
                    [_pad_nodes(kv1), _pad_nodes(kv2)],
                    [srcS, srcD], [dstS, dstD], zeros_pad)
    a1p, a2p = _acc_split(accs[0]), _acc_split(accs[1])
    outs = _tc_level(None, (a1p[0], a1p[1], s1), (a2p[0], a2p[1], s2),
                     [(wl3[0][0], wl3[0][1], 'c12'),
                      (wl1[0][0], wl1[0][1], 'c12'),
                      (wl2[0][0], wl2[0][1], 'c21')])
    p3, p1, p2 = outs[0:3], outs[3:6], outs[6:9]

    xls = []
    for l in range(4):
        accs = _sc_attn(
            [_pad_nodes(p3[0]), _pad_nodes(p1[0]), _pad_nodes(p2[0])],
            [_pad_nodes(p3[1]), _pad_nodes(p1[1]), _pad_nodes(p2[1])],
            [srcE, srcS, srcD], [dstE, dstS, dstD], zeros_pad)
        a3p, a1p, a2p = (_acc_split(accs[0]), _acc_split(accs[1]),
                         _acc_split(accs[2]))
        if l < 3:
            projs = [(wl3[l + 1][0], wl3[l + 1][1], 'c12'),
                     (wl1[l + 1][0], wl1[l + 1][1], 'c12'),
                     (wl2[l + 1][0], wl2[l + 1][1], 'c21')]
        else:
            projs = [(wl3[4][0], wl3[4][1], 'c12')]
        outs = _tc_level((a3p[0], a3p[1], p3[2]),
                         (a1p[0], a1p[1], p1[2]),
                         (a2p[0], a2p[1], p2[2]), projs)
        xls.append(outs[0])
        if l < 3:
            p3, p1, p2 = outs[1:4], outs[4:7], outs[7:10]
        else:
            p3 = outs[1:4]

    accs = _sc_attn([_pad_nodes(p3[0])], [_pad_nodes(p3[1])],
                    [srcE], [dstE], zeros_pad)
    a3p = _acc_split(accs[0])
    return _tc_final(a3p[0], a3p[1], p3[2], xls, fc_W, fc_b)


# speedup vs baseline: 48.1745x; 2.1534x over previous
import math

import jax
import jax.numpy as jnp
from jax import lax
from jax.experimental import pallas as pl
from jax.experimental.pallas import tpu as pltpu
from jax.experimental.pallas import tpu_sc as plsc

_N = 10000
_E = 640000
_D = 20
_NP = 10240
_C = 512
_SUB = 4
_CHUNKS = 40
_EP = 32 * _CHUNKS * _C
_RPT = _NP // 16
_INV = 1.0 / math.sqrt(20.0)
_f32 = jnp.float32


def _sc_attn(qs, kvs, srcs, dsts, zeros_pad):
    nconv = len(qs)
    mesh = plsc.VectorSubcoreMesh(core_axis_name="c", subcore_axis_name="s")
    scratch = [pltpu.VMEM_SHARED((_NP, 24), _f32)]
    scratch += [
        pltpu.VMEM((2, _SUB, 128), jnp.int32),
        pltpu.VMEM((2, _SUB, 128), jnp.int32),
        pltpu.VMEM((2, _C, 24), _f32),
        pltpu.VMEM((2, _C, 48), _f32),
        pltpu.VMEM((_C, 24), _f32),
        pltpu.SemaphoreType.DMA,
        pltpu.SemaphoreType.DMA,
    ]

    def body(*refs):
        ins = refs[: 4 * nconv + 1]
        outs = refs[4 * nconv + 1: 5 * nconv + 1]
        scr = refs[5 * nconv + 1:]
        acc = scr[0]
        dsti, srci, qbuf, kvbuf, obuf, sem_i, sem_g = scr[1:]
        c = lax.axis_index("c")
        s = lax.axis_index("s")
        g = c * 16 + s
        zeros_ref = ins[4 * nconv]
        pltpu.sync_copy(zeros_ref.at[pl.ds(s * _RPT, _RPT)],
                        acc.at[pl.ds(s * _RPT, _RPT)])
        plsc.subcore_barrier()
        iota16 = lax.iota(jnp.int32, 16)
        for v in range(nconv):
            qh, kvh, srch, dsth = ins[4 * v: 4 * v + 4]

            def idx_cp(i, slot, srch=srch, dsth=dsth):
                row = (g * _CHUNKS + i) * _SUB
                return [pltpu.make_async_copy(
                    hm.at[pl.ds(row, _SUB)], buf.at[slot], sem_i)
                    for hm, buf in ((srch, srci), (dsth, dsti))]

            def gath(slot, qh=qh, kvh=kvh):
                cps = []
                for j in range(_SUB):
                    cps.append(pltpu.make_async_copy(
                        qh.at[dsti.at[slot, j]],
                        qbuf.at[slot, pl.ds(j * 128, 128)], sem_g))
                    cps.append(pltpu.make_async_copy(
                        kvh.at[srci.at[slot, j]],
                        kvbuf.at[slot, pl.ds(j * 128, 128)], sem_g))
                return cps

            def compute_scatter(slot):
                qv = qbuf.at[slot]
                kvv = kvbuf.at[slot]

                def group(gi, cc):
                    eids = gi * 16 + iota16
                    a = jnp.zeros((16,), _f32)
                    for d in range(_D):
                        dv = jnp.full((16,), d, jnp.int32)
                        a = a + (plsc.load_gather(qv, [eids, dv])
                                 * plsc.load_gather(kvv, [eids, dv]))
                    e = jnp.exp(a * _INV)
                    for d in range(_D):
                        vd = plsc.load_gather(
                            kvv, [eids, jnp.full((16,), 24 + d, jnp.int32)])
                        plsc.store_scatter(
                            obuf, [eids, jnp.full((16,), d, jnp.int32)],
                            vd * e)
                    plsc.store_scatter(
                        obuf, [eids, jnp.full((16,), _D, jnp.int32)], e)
                    return cc

                lax.fori_loop(0, _C // 16, group, 0)
                for j in range(_SUB):
                    pltpu.sync_copy(obuf.at[pl.ds(j * 128, 128)],
                                    acc.at[dsti.at[slot, j]], add=True)

            for cp in idx_cp(0, 0):
                cp.start()
            for cp in idx_cp(0, 0):
                cp.wait()
            for cp in gath(0):
                cp.start()
            for cp in idx_cp(1, 1):
                cp.start()

            def pipe_half(i, slot, nslot):
                for cp in gath(slot):
                    cp.wait()

                @pl.when(i + 1 < _CHUNKS)
                def _():
                    for cp in idx_cp(i + 1, nslot):
                        cp.wait()
                    for cp in gath(nslot):
                        cp.start()
                compute_scatter(slot)

                @pl.when(i + 2 < _CHUNKS)
                def _():
                    for cp in idx_cp(i + 2, slot):
                        cp.start()

            def pair(ii, carry):
                pipe_half(2 * ii, 0, 1)
                pipe_half(2 * ii + 1, 1, 0)
                return carry

            lax.fori_loop(0, _CHUNKS // 2, pair, 0)
            plsc.subcore_barrier()
            pltpu.sync_copy(acc.at[pl.ds(s * _RPT, _RPT)],
                            outs[v].at[c, pl.ds(s * _RPT, _RPT)])
            if v + 1 < nconv:
                pltpu.sync_copy(zeros_ref.at[pl.ds(s * _RPT, _RPT)],
                                acc.at[pl.ds(s * _RPT, _RPT)])
                plsc.subcore_barrier()

    fn = pl.kernel(
        body,
        out_type=[jax.ShapeDtypeStruct((2, _NP, 24), _f32)] * nconv,
        mesh=mesh,
        scratch_types=scratch,
        compiler_params=pltpu.CompilerParams(
            needs_layout_passes=False, use_tc_tiling_on_sc=False),
    )
    args = []
    for i in range(nconv):
        args += [qs[i], kvs[i], srcs[i], dsts[i]]
    args.append(zeros_pad)
    res = fn(*args)
    if not isinstance(res, (list, tuple)):
        res = [res]
    return list(res)


def _lr(z):
    return jnp.where(z >= 0, z, 0.01 * z)


def _pre(a0, a1, xs):
    u = a0[:, 0:20] + a1[:, 0:20]
    ssum = a0[:, 20:21] + a1[:, 20:21]
    return u / (ssum + 1e-16) + xs


def _norm_lr(pre, stats):
    mu = stats[0:1, :] * (1.0 / _N)
    var = stats[1:2, :] * (1.0 / _N) - mu * mu
    return _lr((pre - mu) * lax.rsqrt(var + 1e-5))


def _split_proj(o, base, q_ref, kv_ref, xs_ref):
    n = o.shape[0]
    z4 = jnp.zeros((n, 4), _f32)
    q = o[:, base: base + 20]
    k = o[:, base + 20: base + 40]
    v = o[:, base + 40: base + 60]
    s = o[:, base + 60: base + 80]
    q_ref[...] = jnp.concatenate([q, z4], axis=1)
    kv_ref[...] = jnp.concatenate([k, z4, v, z4], axis=1)
    xs_ref[...] = s


def _tc_first(features, wcat, bcat):
    R = 1000

    def body(x_ref, w_ref, b_ref, q1, kv1, s1, q2, kv2, s2):
        o = jnp.dot(x_ref[...], w_ref[...],
                    preferred_element_type=_f32) + b_ref[...]
        _split_proj(o, 0, q1, kv1, s1)
        _split_proj(o, 80, q2, kv2, s2)

    outs = [jax.ShapeDtypeStruct((_N, 24), _f32),
            jax.ShapeDtypeStruct((_N, 48), _f32),
            jax.ShapeDtypeStruct((_N, 20), _f32)] * 2
    ospec = [pl.BlockSpec((R, 24), lambda i: (i, 0)),
             pl.BlockSpec((R, 48), lambda i: (i, 0)),
             pl.BlockSpec((R, 20), lambda i: (i, 0))] * 2
    return pl.pallas_call(
        body,
        grid=(_N // R,),
        in_specs=[pl.BlockSpec((R, 2000), lambda i: (i, 0)),
                  pl.BlockSpec((2000, 160), lambda i: (0, 0)),
                  pl.BlockSpec((1, 160), lambda i: (0, 0))],
        out_specs=ospec,
        out_shape=outs,
    )(features, wcat, bcat)


_NB = 10
_R = _N // _NB


def _tc_level(cC, cA, cB, projs):
    has_c = cC is not None
    nproj = len(projs)
    ncomb = 3 if has_c else 2

    def body(*refs):
        ins = refs[: 3 * ncomb + 2 * nproj]
        orefs = refs[3 * ncomb + 2 * nproj: 3 * ncomb + 2 * nproj
                     + (1 if has_c else 0) + 3 * nproj]
        scr = refs[3 * ncomb + 2 * nproj + (1 if has_c else 0) + 3 * nproj:]
        pres = scr[:ncomb]
        stats = scr[ncomb: 2 * ncomb]
        wrefs = ins[3 * ncomb:]
        phase = pl.program_id(0)
        j = pl.program_id(1)

        @pl.when(phase == 0)
        def _():
            for ci in range(ncomb):
                a0, a1, xs = ins[3 * ci: 3 * ci + 3]
                p = _pre(a0[...], a1[...], xs[...])
                pres[ci][pl.ds(j * _R, _R), :] = p
                st = jnp.concatenate(
                    [jnp.sum(p, axis=0, keepdims=True),
                     jnp.sum(p * p, axis=0, keepdims=True)], axis=0)

                @pl.when(j == 0)
                def _(ci=ci, st=st):
                    stats[ci][...] = st

                @pl.when(j > 0)
                def _(ci=ci, st=st):
                    stats[ci][...] = stats[ci][...] + st

        @pl.when(phase == 1)
        def _():
            xall = [_norm_lr(pres[ci][pl.ds(j * _R, _R), :],
                             stats[ci][...]) for ci in range(ncomb)]
            oi = 0
            if has_c:
                orefs[0][...] = xall[0]
                oi = 1
            x1, x2 = xall[-2], xall[-1]
            x12 = jnp.concatenate([x1, x2], axis=1)
            x21 = jnp.concatenate([x2, x1], axis=1)
            for p in range(nproj):
                xin = x12 if projs[p][2] == 'c12' else x21
                o = jnp.dot(xin, wrefs[2 * p][...],
                            preferred_element_type=_f32) + wrefs[2 * p + 1][...]
                _split_proj(o, 0, orefs[oi], orefs[oi + 1], orefs[oi + 2])
                oi += 3

    args = []
    if has_c:
        args += list(cC)
    args += list(cA) + list(cB)
    for w, b, _k in projs:
        args += [w, b]
    blk = lambda r, c: pl.BlockSpec((r, c), lambda p, j: (j, 0))
    full = lambda r, c: pl.BlockSpec((r, c), lambda p, j: (0, 0))
    in_specs = []
    for _ in range(ncomb):
        in_specs += [blk(_R, 24), blk(_R, 24), blk(_R, 20)]
    for _ in range(nproj):
        in_specs += [full(40, 80), full(1, 80)]
    outs = ([jax.ShapeDtypeStruct((_N, 20), _f32)] if has_c else [])
    out_specs = ([blk(_R, 20)] if has_c else [])
    for _ in range(nproj):
        outs += [jax.ShapeDtypeStruct((_N, 24), _f32),
                 jax.ShapeDtypeStruct((_N, 48), _f32),
                 jax.ShapeDtypeStruct((_N, 20), _f32)]
        out_specs += [blk(_R, 24), blk(_R, 48), blk(_R, 20)]
    scratch = ([pltpu.VMEM((_N, 20), _f32)] * ncomb
               + [pltpu.VMEM((2, 20), _f32)] * ncomb)
    return pl.pallas_call(
        body, grid=(2, _NB), in_specs=in_specs, out_specs=out_specs,
        out_shape=outs, scratch_shapes=scratch,
    )(*args)


def _tc_final(a30, a31, xs3, xls, fc_w, fc_b):
    def body(a0, a1, x3, x0, x1, x2, x3l, wr, br, out, pre_s, stat_s):
        phase = pl.program_id(0)
        j = pl.program_id(1)

        @pl.when(phase == 0)
        def _():
            p = _pre(a0[...], a1[...], x3[...])
            pre_s[pl.ds(j * _R, _R), :] = p
            st = jnp.concatenate(
                [jnp.sum(p, axis=0, keepdims=True),
                 jnp.sum(p * p, axis=0, keepdims=True)], axis=0)

            @pl.when(j == 0)
            def _():
                stat_s[...] = st

            @pl.when(j > 0)
            def _():
                stat_s[...] = stat_s[...] + st

        @pl.when(phase == 1)
        def _():
            xl4 = _norm_lr(pre_s[pl.ds(j * _R, _R), :], stat_s[...])
            fc = jnp.concatenate(
                [x0[...], x1[...], x2[...], x3l[...], xl4], axis=1)
            out[...] = jnp.dot(fc, wr[...],
                               preferred_element_type=_f32) + br[...]

    blk = lambda r, c: pl.BlockSpec((r, c), lambda p, j: (j, 0))
    full = lambda r, c: pl.BlockSpec((r, c), lambda p, j: (0, 0))
    return pl.pallas_call(
        body, grid=(2, _NB),
        in_specs=[blk(_R, 24), blk(_R, 24), blk(_R, 20)]
        + [blk(_R, 20)] * 4 + [full(100, 2), full(1, 2)],
        out_specs=blk(_R, 2),
        out_shape=jax.ShapeDtypeStruct((_N, 2), _f32),
        scratch_shapes=[pltpu.VMEM((_N, 20), _f32),
                        pltpu.VMEM((2, 20), _f32)],
    )(a30, a31, xs3, xls[0], xls[1], xls[2], xls[3], fc_w,
      fc_b[None, :])


def _prep_edges(ei):
    pad = jnp.full((_EP - _E,), _N, jnp.int32)
    src = jnp.concatenate([ei[0].astype(jnp.int32), pad]).reshape(-1, 128)
    dst = jnp.concatenate([ei[1].astype(jnp.int32), pad]).reshape(-1, 128)
    return src, dst


def _pad_nodes(t):
    return jnp.concatenate(
        [t, jnp.zeros((_NP - _N, t.shape[1]), _f32)], axis=0)


def _wcat(wq, bq, wk, bk, wv, bv, ws, bs):
    w = jnp.concatenate([wq, wk, wv, ws], axis=1)
    b = jnp.concatenate([bq, bk, bv, bs])[None, :]
    return w, b


def _acc_split(acc):
    return acc[0, :_N, :], acc[1, :_N, :]


def kernel(features, same_index, diff_index, edge_index, same_index2, diff_index2,
           c1_Wq0, c1_bq0, c1_Wk0, c1_bk0, c1_Wv0, c1_bv0, c1_Ws0, c1_bs0,
           c1_Wq, c1_bq, c1_Wk, c1_bk, c1_Wv, c1_bv, c1_Ws, c1_bs,
           c2_Wq0, c2_bq0, c2_Wk0, c2_bk0, c2_Wv0, c2_bv0, c2_Ws0, c2_bs0,
           c2_Wq, c2_bq, c2_Wk, c2_bk, c2_Wv, c2_bv, c2_Ws, c2_bs,
           c3_Wq, c3_bq, c3_Wk, c3_bk, c3_Wv, c3_bv, c3_Ws, c3_bs,
           fc_W, fc_b):
    w1, b1 = _wcat(c1_Wq0, c1_bq0, c1_Wk0, c1_bk0, c1_Wv0, c1_bv0, c1_Ws0, c1_bs0)
    w2, b2 = _wcat(c2_Wq0, c2_bq0, c2_Wk0, c2_bk0, c2_Wv0, c2_bv0, c2_Ws0, c2_bs0)
    wcat = jnp.concatenate([w1, w2], axis=1)
    bcat = jnp.concatenate([b1, b2], axis=1)
    wl1 = [_wcat(c1_Wq[l], c1_bq[l], c1_Wk[l], c1_bk[l],
                 c1_Wv[l], c1_bv[l], c1_Ws[l], c1_bs[l]) for l in range(4)]
    wl2 = [_wcat(c2_Wq[l], c2_bq[l], c2_Wk[l], c2_bk[l],
                 c2_Wv[l], c2_bv[l], c2_Ws[l], c2_bs[l]) for l in range(4)]
    wl3 = [_wcat(c3_Wq[l], c3_bq[l], c3_Wk[l], c3_bk[l],
                 c3_Wv[l], c3_bv[l], c3_Ws[l], c3_bs[l]) for l in range(5)]

    srcS, dstS = _prep_edges(same_index2)
    srcD, dstD = _prep_edges(diff_index2)
    srcE, dstE = _prep_edges(edge_index)
    zeros_pad = jnp.zeros((_NP, 24), _f32)

    q1, kv1, s1, q2, kv2, s2 = _tc_first(features, wcat, bcat)
    accs = _sc_attn([_pad_nodes(q1), _pad_nodes(q2)],
                    [_pad_nodes(kv1), _pad_nodes(kv2)],
                    [srcS, srcD], [dstS, dstD], zeros_pad)
    a1p, a2p = _acc_split(accs[0]), _acc_split(accs[1])
    outs = _tc_level(None, (a1p[0], a1p[1], s1), (a2p[0], a2p[1], s2),
                     [(wl3[0][0], wl3[0][1], 'c12'),
                      (wl1[0][0], wl1[0][1], 'c12'),
                      (wl2[0][0], wl2[0][1], 'c21')])
    p3, p1, p2 = outs[0:3], outs[3:6], outs[6:9]

    xls = []
    for l in range(4):
        accs = _sc_attn(
            [_pad_nodes(p3[0]), _pad_nodes(p1[0]), _pad_nodes(p2[0])],
            [_pad_nodes(p3[1]), _pad_nodes(p1[1]), _pad_nodes(p2[1])],
            [srcE, srcS, srcD], [dstE, dstS, dstD], zeros_pad)
        a3p, a1p, a2p = (_acc_split(accs[0]), _acc_split(accs[1]),
                         _acc_split(accs[2]))
        if l < 3:
            projs = [(wl3[l + 1][0], wl3[l + 1][1], 'c12'),
                     (wl1[l + 1][0], wl1[l + 1][1], 'c12'),
                     (wl2[l + 1][0], wl2[l + 1][1], 'c21')]
        else:
            projs = [(wl3[4][0], wl3[4][1], 'c12')]
        outs = _tc_level((a3p[0], a3p[1], p3[2]),
                         (a1p[0], a1p[1], p1[2]),
                         (a2p[0], a2p[1], p2[2]), projs)
        xls.append(outs[0])
        if l < 3:
            p3, p1, p2 = outs[1:4], outs[4:7], outs[7:10]
        else:
            p3 = outs[1:4]

    accs = _sc_attn([_pad_nodes(p3[0])], [_pad_nodes(p3[1])],
                    [srcE], [dstE], zeros_pad)
    a3p = _acc_split(accs[0])
    return _tc_final(a3p[0], a3p[1], p3[2], xls, fc_W, fc_b)


# speedup vs baseline: 51.3706x; 1.0663x over previous
import math

import jax
import jax.numpy as jnp
from jax import lax
from jax.experimental import pallas as pl
from jax.experimental.pallas import tpu as pltpu
from jax.experimental.pallas import tpu_sc as plsc

_N = 10000
_E = 640000
_D = 20
_NP = 10240
_C = 512
_SUB = 4
_CHUNKS = 40
_EP = 32 * _CHUNKS * _C
_RPT = _NP // 16
_INV = 1.0 / math.sqrt(20.0)
_f32 = jnp.float32


def _sc_attn(qs, kvs, srcs, dsts, zeros_pad):
    nconv = len(qs)
    mesh = plsc.VectorSubcoreMesh(core_axis_name="c", subcore_axis_name="s")
    scratch = [pltpu.VMEM_SHARED((_NP, 24), _f32)]
    scratch += [
        pltpu.VMEM((2, _SUB, 128), jnp.int32),
        pltpu.VMEM((2, _SUB, 128), jnp.int32),
        pltpu.VMEM((2, _C, 24), _f32),
        pltpu.VMEM((2, _C, 48), _f32),
        pltpu.VMEM((_C, 24), _f32),
        pltpu.SemaphoreType.DMA,
        pltpu.SemaphoreType.DMA,
    ]

    def body(*refs):
        ins = refs[: 4 * nconv + 1]
        outs = refs[4 * nconv + 1: 5 * nconv + 1]
        scr = refs[5 * nconv + 1:]
        acc = scr[0]
        dsti, srci, qbuf, kvbuf, obuf, sem_i, sem_g = scr[1:]
        c = lax.axis_index("c")
        s = lax.axis_index("s")
        g = c * 16 + s
        zeros_ref = ins[4 * nconv]
        pltpu.sync_copy(zeros_ref.at[pl.ds(s * _RPT, _RPT)],
                        acc.at[pl.ds(s * _RPT, _RPT)])
        plsc.subcore_barrier()
        iota16 = lax.iota(jnp.int32, 16)
        for v in range(nconv):
            qh, kvh, srch, dsth = ins[4 * v: 4 * v + 4]

            def idx_cp(i, slot, srch=srch, dsth=dsth):
                row = (g * _CHUNKS + i) * _SUB
                return [pltpu.make_async_copy(
                    hm.at[pl.ds(row, _SUB)], buf.at[slot], sem_i)
                    for hm, buf in ((srch, srci), (dsth, dsti))]

            def gath(slot, qh=qh, kvh=kvh):
                cps = []
                for j in range(_SUB):
                    cps.append(pltpu.make_async_copy(
                        qh.at[dsti.at[slot, j]],
                        qbuf.at[slot, pl.ds(j * 128, 128)], sem_g))
                    cps.append(pltpu.make_async_copy(
                        kvh.at[srci.at[slot, j]],
                        kvbuf.at[slot, pl.ds(j * 128, 128)], sem_g))
                return cps

            def compute_scatter(slot):
                qv = qbuf.at[slot]
                kvv = kvbuf.at[slot]

                @plsc.parallel_loop(0, _C // 16)
                def group(gi):
                    eids = gi * 16 + iota16
                    a = jnp.zeros((16,), _f32)
                    for d in range(_D):
                        dv = jnp.full((16,), d, jnp.int32)
                        a = a + (plsc.load_gather(qv, [eids, dv])
                                 * plsc.load_gather(kvv, [eids, dv]))
                    e = jnp.exp(a * _INV)
                    for d in range(_D):
                        vd = plsc.load_gather(
                            kvv, [eids, jnp.full((16,), 24 + d, jnp.int32)])
                        plsc.store_scatter(
                            obuf, [eids, jnp.full((16,), d, jnp.int32)],
                            vd * e)
                    plsc.store_scatter(
                        obuf, [eids, jnp.full((16,), _D, jnp.int32)], e)
                for j in range(_SUB):
                    pltpu.sync_copy(obuf.at[pl.ds(j * 128, 128)],
                                    acc.at[dsti.at[slot, j]], add=True)

            for cp in idx_cp(0, 0):
                cp.start()
            for cp in idx_cp(0, 0):
                cp.wait()
            for cp in gath(0):
                cp.start()
            for cp in idx_cp(1, 1):
                cp.start()

            def pipe_half(i, slot, nslot):
                for cp in gath(slot):
                    cp.wait()

                @pl.when(i + 1 < _CHUNKS)
                def _():
                    for cp in idx_cp(i + 1, nslot):
                        cp.wait()
                    for cp in gath(nslot):
                        cp.start()
                compute_scatter(slot)

                @pl.when(i + 2 < _CHUNKS)
                def _():
                    for cp in idx_cp(i + 2, slot):
                        cp.start()

            def pair(ii, carry):
                pipe_half(2 * ii, 0, 1)
                pipe_half(2 * ii + 1, 1, 0)
                return carry

            lax.fori_loop(0, _CHUNKS // 2, pair, 0)
            plsc.subcore_barrier()
            pltpu.sync_copy(acc.at[pl.ds(s * _RPT, _RPT)],
                            outs[v].at[c, pl.ds(s * _RPT, _RPT)])
            if v + 1 < nconv:
                pltpu.sync_copy(zeros_ref.at[pl.ds(s * _RPT, _RPT)],
                                acc.at[pl.ds(s * _RPT, _RPT)])
                plsc.subcore_barrier()

    fn = pl.kernel(
        body,
        out_type=[jax.ShapeDtypeStruct((2, _NP, 24), _f32)] * nconv,
        mesh=mesh,
        scratch_types=scratch,
        compiler_params=pltpu.CompilerParams(
            needs_layout_passes=False, use_tc_tiling_on_sc=False),
    )
    args = []
    for i in range(nconv):
        args += [qs[i], kvs[i], srcs[i], dsts[i]]
    args.append(zeros_pad)
    res = fn(*args)
    if not isinstance(res, (list, tuple)):
        res = [res]
    return list(res)


def _lr(z):
    return jnp.where(z >= 0, z, 0.01 * z)


def _pre(a0, a1, xs):
    u = a0[:, 0:20] + a1[:, 0:20]
    ssum = a0[:, 20:21] + a1[:, 20:21]
    return u / (ssum + 1e-16) + xs


def _norm_lr(pre, stats):
    mu = stats[0:1, :] * (1.0 / _N)
    var = stats[1:2, :] * (1.0 / _N) - mu * mu
    return _lr((pre - mu) * lax.rsqrt(var + 1e-5))


def _split_proj(o, base, q_ref, kv_ref, xs_ref):
    n = o.shape[0]
    z4 = jnp.zeros((n, 4), _f32)
    q = o[:, base: base + 20]
    k = o[:, base + 20: base + 40]
    v = o[:, base + 40: base + 60]
    s = o[:, base + 60: base + 80]
    q_ref[...] = jnp.concatenate([q, z4], axis=1)
    kv_ref[...] = jnp.concatenate([k, z4, v, z4], axis=1)
    xs_ref[...] = s


def _tc_first(features, wcat, bcat):
    R = 1000

    def body(x_ref, w_ref, b_ref, q1, kv1, s1, q2, kv2, s2):
        o = jnp.dot(x_ref[...], w_ref[...],
                    preferred_element_type=_f32) + b_ref[...]
        _split_proj(o, 0, q1, kv1, s1)
        _split_proj(o, 80, q2, kv2, s2)

    outs = [jax.ShapeDtypeStruct((_N, 24), _f32),
            jax.ShapeDtypeStruct((_N, 48), _f32),
            jax.ShapeDtypeStruct((_N, 20), _f32)] * 2
    ospec = [pl.BlockSpec((R, 24), lambda i: (i, 0)),
             pl.BlockSpec((R, 48), lambda i: (i, 0)),
             pl.BlockSpec((R, 20), lambda i: (i, 0))] * 2
    return pl.pallas_call(
        body,
        grid=(_N // R,),
        in_specs=[pl.BlockSpec((R, 2000), lambda i: (i, 0)),
                  pl.BlockSpec((2000, 160), lambda i: (0, 0)),
                  pl.BlockSpec((1, 160), lambda i: (0, 0))],
        out_specs=ospec,
        out_shape=outs,
    )(features, wcat, bcat)


_NB = 10
_R = _N // _NB


def _tc_level(cC, cA, cB, projs):
    has_c = cC is not None
    nproj = len(projs)
    ncomb = 3 if has_c else 2

    def body(*refs):
        ins = refs[: 3 * ncomb + 2 * nproj]
        orefs = refs[3 * ncomb + 2 * nproj: 3 * ncomb + 2 * nproj
                     + (1 if has_c else 0) + 3 * nproj]
        scr = refs[3 * ncomb + 2 * nproj + (1 if has_c else 0) + 3 * nproj:]
        pres = scr[:ncomb]
        stats = scr[ncomb: 2 * ncomb]
        wrefs = ins[3 * ncomb:]
        phase = pl.program_id(0)
        j = pl.program_id(1)

        @pl.when(phase == 0)
        def _():
            for ci in range(ncomb):
                a0, a1, xs = ins[3 * ci: 3 * ci + 3]
                p = _pre(a0[...], a1[...], xs[...])
                pres[ci][pl.ds(j * _R, _R), :] = p
                st = jnp.concatenate(
                    [jnp.sum(p, axis=0, keepdims=True),
                     jnp.sum(p * p, axis=0, keepdims=True)], axis=0)

                @pl.when(j == 0)
                def _(ci=ci, st=st):
                    stats[ci][...] = st

                @pl.when(j > 0)
                def _(ci=ci, st=st):
                    stats[ci][...] = stats[ci][...] + st

        @pl.when(phase == 1)
        def _():
            xall = [_norm_lr(pres[ci][pl.ds(j * _R, _R), :],
                             stats[ci][...]) for ci in range(ncomb)]
            oi = 0
            if has_c:
                orefs[0][...] = xall[0]
                oi = 1
            x1, x2 = xall[-2], xall[-1]
            x12 = jnp.concatenate([x1, x2], axis=1)
            x21 = jnp.concatenate([x2, x1], axis=1)
            for p in range(nproj):
                xin = x12 if projs[p][2] == 'c12' else x21
                o = jnp.dot(xin, wrefs[2 * p][...],
                            preferred_element_type=_f32) + wrefs[2 * p + 1][...]
                _split_proj(o, 0, orefs[oi], orefs[oi + 1], orefs[oi + 2])
                oi += 3

    args = []
    if has_c:
        args += list(cC)
    args += list(cA) + list(cB)
    for w, b, _k in projs:
        args += [w, b]
    blk = lambda r, c: pl.BlockSpec((r, c), lambda p, j: (j, 0))
    full = lambda r, c: pl.BlockSpec((r, c), lambda p, j: (0, 0))
    in_specs = []
    for _ in range(ncomb):
        in_specs += [blk(_R, 24), blk(_R, 24), blk(_R, 20)]
    for _ in range(nproj):
        in_specs += [full(40, 80), full(1, 80)]
    outs = ([jax.ShapeDtypeStruct((_N, 20), _f32)] if has_c else [])
    out_specs = ([blk(_R, 20)] if has_c else [])
    for _ in range(nproj):
        outs += [jax.ShapeDtypeStruct((_N, 24), _f32),
                 jax.ShapeDtypeStruct((_N, 48), _f32),
                 jax.ShapeDtypeStruct((_N, 20), _f32)]
        out_specs += [blk(_R, 24), blk(_R, 48), blk(_R, 20)]
    scratch = ([pltpu.VMEM((_N, 20), _f32)] * ncomb
               + [pltpu.VMEM((2, 20), _f32)] * ncomb)
    return pl.pallas_call(
        body, grid=(2, _NB), in_specs=in_specs, out_specs=out_specs,
        out_shape=outs, scratch_shapes=scratch,
    )(*args)


def _tc_final(a30, a31, xs3, xls, fc_w, fc_b):
    def body(a0, a1, x3, x0, x1, x2, x3l, wr, br, out, pre_s, stat_s):
        phase = pl.program_id(0)
        j = pl.program_id(1)

        @pl.when(phase == 0)
        def _():
            p = _pre(a0[...], a1[...], x3[...])
            pre_s[pl.ds(j * _R, _R), :] = p
            st = jnp.concatenate(
                [jnp.sum(p, axis=0, keepdims=True),
                 jnp.sum(p * p, axis=0, keepdims=True)], axis=0)

            @pl.when(j == 0)
            def _():
                stat_s[...] = st

            @pl.when(j > 0)
            def _():
                stat_s[...] = stat_s[...] + st

        @pl.when(phase == 1)
        def _():
            xl4 = _norm_lr(pre_s[pl.ds(j * _R, _R), :], stat_s[...])
            fc = jnp.concatenate(
                [x0[...], x1[...], x2[...], x3l[...], xl4], axis=1)
            out[...] = jnp.dot(fc, wr[...],
                               preferred_element_type=_f32) + br[...]

    blk = lambda r, c: pl.BlockSpec((r, c), lambda p, j: (j, 0))
    full = lambda r, c: pl.BlockSpec((r, c), lambda p, j: (0, 0))
    return pl.pallas_call(
        body, grid=(2, _NB),
        in_specs=[blk(_R, 24), blk(_R, 24), blk(_R, 20)]
        + [blk(_R, 20)] * 4 + [full(100, 2), full(1, 2)],
        out_specs=blk(_R, 2),
        out_shape=jax.ShapeDtypeStruct((_N, 2), _f32),
        scratch_shapes=[pltpu.VMEM((_N, 20), _f32),
                        pltpu.VMEM((2, 20), _f32)],
    )(a30, a31, xs3, xls[0], xls[1], xls[2], xls[3], fc_w,
      fc_b[None, :])


def _prep_edges(ei):
    pad = jnp.full((_EP - _E,), _N, jnp.int32)
    src = jnp.concatenate([ei[0].astype(jnp.int32), pad]).reshape(-1, 128)
    dst = jnp.concatenate([ei[1].astype(jnp.int32), pad]).reshape(-1, 128)
    return src, dst


def _pad_nodes(t):
    return jnp.concatenate(
        [t, jnp.zeros((_NP - _N, t.shape[1]), _f32)], axis=0)


def _wcat(wq, bq, wk, bk, wv, bv, ws, bs):
    w = jnp.concatenate([wq, wk, wv, ws], axis=1)
    b = jnp.concatenate([bq, bk, bv, bs])[None, :]
    return w, b


def _acc_split(acc):
    return acc[0, :_N, :], acc[1, :_N, :]


def kernel(features, same_index, diff_index, edge_index, same_index2, diff_index2,
           c1_Wq0, c1_bq0, c1_Wk0, c1_bk0, c1_Wv0, c1_bv0, c1_Ws0, c1_bs0,
           c1_Wq, c1_bq, c1_Wk, c1_bk, c1_Wv, c1_bv, c1_Ws, c1_bs,
           c2_Wq0, c2_bq0, c2_Wk0, c2_bk0, c2_Wv0, c2_bv0, c2_Ws0, c2_bs0,
           c2_Wq, c2_bq, c2_Wk, c2_bk, c2_Wv, c2_bv, c2_Ws, c2_bs,
           c3_Wq, c3_bq, c3_Wk, c3_bk, c3_Wv, c3_bv, c3_Ws, c3_bs,
           fc_W, fc_b):
    w1, b1 = _wcat(c1_Wq0, c1_bq0, c1_Wk0, c1_bk0, c1_Wv0, c1_bv0, c1_Ws0, c1_bs0)
    w2, b2 = _wcat(c2_Wq0, c2_bq0, c2_Wk0, c2_bk0, c2_Wv0, c2_bv0, c2_Ws0, c2_bs0)
    wcat = jnp.concatenate([w1, w2], axis=1)
    bcat = jnp.concatenate([b1, b2], axis=1)
    wl1 = [_wcat(c1_Wq[l], c1_bq[l], c1_Wk[l], c1_bk[l],
                 c1_Wv[l], c1_bv[l], c1_Ws[l], c1_bs[l]) for l in range(4)]
    wl2 = [_wcat(c2_Wq[l], c2_bq[l], c2_Wk[l], c2_bk[l],
                 c2_Wv[l], c2_bv[l], c2_Ws[l], c2_bs[l]) for l in range(4)]
    wl3 = [_wcat(c3_Wq[l], c3_bq[l], c3_Wk[l], c3_bk[l],
                 c3_Wv[l], c3_bv[l], c3_Ws[l], c3_bs[l]) for l in range(5)]

    srcS, dstS = _prep_edges(same_index2)
    srcD, dstD = _prep_edges(diff_index2)
    srcE, dstE = _prep_edges(edge_index)
    zeros_pad = jnp.zeros((_NP, 24), _f32)

    q1, kv1, s1, q2, kv2, s2 = _tc_first(features, wcat, bcat)
    accs = _sc_attn([_pad_nodes(q1), _pad_nodes(q2)],
                    [_pad_nodes(kv1), _pad_nodes(kv2)],
                    [srcS, srcD], [dstS, dstD], zeros_pad)
    a1p, a2p = _acc_split(accs[0]), _acc_split(accs[1])
    outs = _tc_level(None, (a1p[0], a1p[1], s1), (a2p[0], a2p[1], s2),
                     [(wl3[0][0], wl3[0][1], 'c12'),
                      (wl1[0][0], wl1[0][1], 'c12'),
                      (wl2[0][0], wl2[0][1], 'c21')])
    p3, p1, p2 = outs[0:3], outs[3:6], outs[6:9]

    xls = []
    for l in range(4):
        accs = _sc_attn(
            [_pad_nodes(p3[0]), _pad_nodes(p1[0]), _pad_nodes(p2[0])],
            [_pad_nodes(p3[1]), _pad_nodes(p1[1]), _pad_nodes(p2[1])],
            [srcE, srcS, srcD], [dstE, dstS, dstD], zeros_pad)
        a3p, a1p, a2p = (_acc_split(accs[0]), _acc_split(accs[1]),
                         _acc_split(accs[2]))
        if l < 3:
            projs = [(wl3[l + 1][0], wl3[l + 1][1], 'c12'),
                     (wl1[l + 1][0], wl1[l + 1][1], 'c12'),
                     (wl2[l + 1][0], wl2[l + 1][1], 'c21')]
        else:
            projs = [(wl3[4][0], wl3[4][1], 'c12')]
        outs = _tc_level((a3p[0], a3p[1], p3[2]),
                         (a1p[0], a1p[1], p1[2]),
                         (a2p[0], a2p[1], p2[2]), projs)
        xls.append(outs[0])
        if l < 3:
            p3, p1, p2 = outs[1:4], outs[4:7], outs[7:10]
        else:
            p3 = outs[1:4]

    accs = _sc_attn([_pad_nodes(p3[0])], [_pad_nodes(p3[1])],
                    [srcE], [dstE], zeros_pad)
    a3p = _acc_split(accs[0])
    return _tc_final(a3p[0], a3p[1], p3[2], xls, fc_W, fc_b)
